# baseline (speedup 1.0000x reference)
import functools

import jax
import jax.numpy as jnp
from jax import lax
from jax.experimental import pallas as pl
from jax.experimental.pallas import tpu as pltpu
from jax.experimental.pallas import tpu_sc as plsc

B = 16384
D = 64
L = 16

_info = plsc.get_sparse_core_info()
NC = _info.num_cores
NS = _info.num_subcores
NW = NC * NS
PW = B // NW
CPR = 128
NCK = PW // CPR
CGRP = CPR // L


def _trans_e_body(h_hbm, r_hbm, t_hbm, ent_hbm, rel_hbm, out_hbm,
                  him, rim, tim, hb, rb, tb, outv, sem0, sem1):
    wid = lax.axis_index("s") * NC + lax.axis_index("c")
    sems = (sem0, sem1)

    pltpu.sync_copy(h_hbm.at[wid], him)
    pltpu.sync_copy(r_hbm.at[wid], rim)
    pltpu.sync_copy(t_hbm.at[wid], tim)

    lane = lax.iota(jnp.int32, L)

    def fire(k, slot):
        base = k * CPR
        sem = sems[slot]

        @plsc.parallel_loop(0, CPR // L)
        def fire_body(q):
            hv16 = him[pl.ds(base + q * L, L)]
            rv16 = rim[pl.ds(base + q * L, L)]
            tv16 = tim[pl.ds(base + q * L, L)]
            for jj in range(L):
                i = q * L + jj
                pltpu.async_copy(ent_hbm.at[hv16[jj]], hb.at[slot].at[i], sem)
                pltpu.async_copy(rel_hbm.at[rv16[jj]], rb.at[slot].at[i], sem)
                pltpu.async_copy(ent_hbm.at[tv16[jj]], tb.at[slot].at[i], sem)

    def drain(slot):
        sem = sems[slot]

        def drain_body(i, _):
            pltpu.make_async_copy(ent_hbm.at[0], hb.at[slot].at[0], sem).wait()
            pltpu.make_async_copy(rel_hbm.at[0], rb.at[slot].at[0], sem).wait()
            pltpu.make_async_copy(ent_hbm.at[0], tb.at[slot].at[0], sem).wait()
            return 0

        lax.fori_loop(0, CPR, drain_body, 0)

    def compute(k, slot):
        base = k * CPR

        def group_body(g, _):
            row = lane + g * L
            acc = jnp.zeros((L,), jnp.float32)
            for c in range(D):
                col = jnp.full((L,), c, jnp.int32)
                hv = plsc.load_gather(hb.at[slot], [row, col])
                rv = plsc.load_gather(rb.at[slot], [row, col])
                tv = plsc.load_gather(tb.at[slot], [row, col])
                acc = acc + jnp.abs(hv + rv - tv)
            outv[pl.ds(base + g * L, L)] = acc
            return 0

        lax.fori_loop(0, CGRP, group_body, 0)

    fire(0, 0)
    for k in range(NCK):
        if k + 1 < NCK:
            fire(k + 1, (k + 1) % 2)
        drain(k % 2)
        compute(k, k % 2)

    pltpu.sync_copy(outv, out_hbm.at[pl.ds(wid * PW, PW)])


@jax.jit
def _trans_e(h, r, t, entity_emb, relation_emb):
    mesh = plsc.VectorSubcoreMesh(core_axis_name="c", subcore_axis_name="s")
    run = functools.partial(
        pl.kernel,
        mesh=mesh,
        compiler_params=pltpu.CompilerParams(needs_layout_passes=False),
        out_type=jax.ShapeDtypeStruct((B,), jnp.float32),
        scratch_types=[
            pltpu.VMEM((PW,), jnp.int32),
            pltpu.VMEM((PW,), jnp.int32),
            pltpu.VMEM((PW,), jnp.int32),
            pltpu.VMEM((2, CPR, D), jnp.float32),
            pltpu.VMEM((2, CPR, D), jnp.float32),
            pltpu.VMEM((2, CPR, D), jnp.float32),
            pltpu.VMEM((PW,), jnp.float32),
            pltpu.SemaphoreType.DMA,
            pltpu.SemaphoreType.DMA,
        ],
    )(_trans_e_body)
    return run(h, r, t, entity_emb, relation_emb)


def kernel(h, r, t, entity_emb, relation_emb):
    h2 = h.astype(jnp.int32).reshape(NW, PW)
    r2 = r.astype(jnp.int32).reshape(NW, PW)
    t2 = t.astype(jnp.int32).reshape(NW, PW)
    return _trans_e(h2, r2, t2, entity_emb, relation_emb)
